# Initial kernel scaffold; baseline (speedup 1.0000x reference)
import jax
import jax.numpy as jnp
from jax.experimental import pallas as pl


def kernel(x):
    raise NotImplementedError("write your pallas kernel here")


# speedup vs baseline: 5.0317x; 5.0317x over previous
import functools

import jax
import jax.numpy as jnp
from jax import lax
from jax.experimental import pallas as pl
from jax.experimental.pallas import tpu as pltpu

B, N, C, K = 8, 2048, 64, 20
R = 256


def _topk_body(x_ref, xt_ref, idx_ref):
    b = pl.program_id(0)
    xr = x_ref[0]
    xt = xt_ref[0]
    d = 2.0 * jnp.dot(xr, xt, preferred_element_type=jnp.float32)
    d = d - jnp.sum(xr * xr, axis=1, keepdims=True)
    d = d - jnp.sum(xt * xt, axis=0, keepdims=True)
    col = lax.broadcasted_iota(jnp.int32, (R, N), 1)
    base = b * N
    for k in range(K):
        m = jnp.max(d, axis=1, keepdims=True)
        sel = jnp.where(d == m, col, N)
        a = jnp.min(sel, axis=1, keepdims=True)
        idx_ref[0, :, k] = a[:, 0] + base
        d = jnp.where(col == a, -jnp.inf, d)


def _topk_call(x, xt):
    return pl.pallas_call(
        _topk_body,
        grid=(B, N // R),
        in_specs=[
            pl.BlockSpec((1, R, C), lambda b, i: (b, i, 0)),
            pl.BlockSpec((1, C, N), lambda b, i: (b, 0, 0)),
        ],
        out_specs=pl.BlockSpec((1, R, K), lambda b, i: (b, i, 0)),
        out_shape=jax.ShapeDtypeStruct((B, N, K), jnp.int32),
    )(x, xt)


def kernel(x):
    xt = jnp.transpose(x, (0, 2, 1))
    idx = _topk_call(x, xt)
    xf = x.reshape(B * N, C)
    edge1 = xf[idx.reshape(-1) % (B * N)].reshape(B, N, K, C)
    edge2 = jnp.broadcast_to(x[:, :, None, :], (B, N, K, C))
    return jnp.concatenate([edge1, edge2], axis=-1)


# speedup vs baseline: 9.4664x; 1.8813x over previous
import functools

import jax
import jax.numpy as jnp
from jax import lax
from jax.experimental import pallas as pl
from jax.experimental.pallas import tpu as pltpu
from jax.experimental.pallas import tpu_sc as plsc

B, N, C, K = 8, 2048, 64, 20
R = 256

NW = 32
RPW = B * N // NW
CH = 32
NCH = RPW // CH
IPC = CH * K // 128


def _topk_body(x_ref, xt_ref, xsqr_ref, xsqc_ref, idx_ref):
    b = pl.program_id(0)
    xr = x_ref[0]
    xt = xt_ref[0]
    d = 2.0 * jnp.dot(xr, xt, preferred_element_type=jnp.float32)
    d = d - xsqr_ref[0]
    d = d - xsqc_ref[0]
    col = lax.broadcasted_iota(jnp.int32, (R, N), 1)
    base = b * N
    for k in range(K):
        m = jnp.max(d, axis=1, keepdims=True)
        sel = jnp.where(d == m, col, N)
        a = jnp.min(sel, axis=1, keepdims=True)
        idx_ref[0, :, k] = a[:, 0] + base
        d = jnp.where(col == a, -jnp.inf, d)


def _topk_call(x, xt, xsq):
    return pl.pallas_call(
        _topk_body,
        grid=(B, N // R),
        in_specs=[
            pl.BlockSpec((1, R, C), lambda b, i: (b, i, 0)),
            pl.BlockSpec((1, C, N), lambda b, i: (b, 0, 0)),
            pl.BlockSpec((1, R, 1), lambda b, i: (b, i, 0)),
            pl.BlockSpec((1, 1, N), lambda b, i: (b, 0, 0)),
        ],
        out_specs=pl.BlockSpec((1, R, K), lambda b, i: (b, i, 0)),
        out_shape=jax.ShapeDtypeStruct((B, N, K), jnp.int32),
    )(x, xt, xsq, xsq.reshape(B, 1, N))


def _sc_gather_body(x_hbm, idx_hbm, out_hbm, idx_v, self_v, comb_v, sem):
    c = lax.axis_index("c")
    s = lax.axis_index("s")
    wid = s * 2 + c

    def chunk(t, carry):
        base = wid * RPW + t * CH
        pltpu.sync_copy(idx_hbm.at[pl.ds(base * K, CH * K)], idx_v)
        copies = [
            pltpu.async_copy(x_hbm.at[idx_v.at[pl.ds(g * 128, 128)]],
                             comb_v.at[pl.ds(g * 128, 128)], sem)
            for g in range(IPC)
        ]
        pltpu.sync_copy(x_hbm.at[pl.ds(base, CH)], self_v)
        for cp in copies:
            cp.wait()

        def rep(r, carry2):
            for q in range(C // 16):
                v = self_v[r, pl.ds(q * 16, 16)]
                for k in range(K):
                    comb_v[r * K + k, pl.ds(C + q * 16, 16)] = v
            return carry2

        lax.fori_loop(0, CH, rep, 0)
        pltpu.sync_copy(comb_v, out_hbm.at[pl.ds(base * K, CH * K)])
        return carry

    lax.fori_loop(0, NCH, chunk, 0)


_sc_gather = functools.partial(
    pl.kernel,
    out_type=jax.ShapeDtypeStruct((B * N * K, 2 * C), jnp.float32),
    mesh=plsc.VectorSubcoreMesh(core_axis_name="c", subcore_axis_name="s"),
    scratch_types=[
        pltpu.VMEM((CH * K,), jnp.int32),
        pltpu.VMEM((CH, 2 * C), jnp.float32),
        pltpu.VMEM((CH * K, 2 * C), jnp.float32),
        pltpu.SemaphoreType.DMA,
    ],
)(_sc_gather_body)


def kernel(x):
    xt = jnp.transpose(x, (0, 2, 1))
    xsq = jnp.sum(x ** 2, axis=2, keepdims=True)
    idx = _topk_call(x, xt, xsq)
    xf = x.reshape(B * N, C)
    xx = jnp.concatenate([xf, xf], axis=1)
    idx2 = idx.reshape(B * N * K)
    out = _sc_gather(xx, idx2)
    return out.reshape(B, N, K, 2 * C)
